# Initial kernel scaffold; baseline (speedup 1.0000x reference)
import jax
import jax.numpy as jnp
from jax.experimental import pallas as pl


def kernel(positions, table):
    raise NotImplementedError("write your pallas kernel here")


# speedup vs baseline: 4.5382x; 4.5382x over previous
import functools

import jax
import jax.numpy as jnp
from jax import lax
from jax.experimental import pallas as pl
from jax.experimental.pallas import tpu as pltpu
from jax.experimental.pallas import tpu_sc as plsc

_NC = 2
_NS = 16
_NW = _NC * _NS

_DIM = 32
_B_TOTAL = 16384 * 200
_IDX_COLS = 128
_IDX_ROWS = _B_TOTAL // _IDX_COLS
_ROWS_PER_W = _IDX_ROWS // _NW
_GROUPS = 8
_CHUNK = _GROUPS * _IDX_COLS
_N_CHUNKS = _ROWS_PER_W // _GROUPS

_mesh = plsc.VectorSubcoreMesh(
    core_axis_name="c", subcore_axis_name="s", num_cores=_NC, num_subcores=_NS
)


@functools.partial(
    pl.kernel,
    out_type=jax.ShapeDtypeStruct((_B_TOTAL, _DIM), jnp.float32),
    mesh=_mesh,
    scratch_types=[
        pltpu.VMEM((_GROUPS, _IDX_COLS), jnp.int32),
        pltpu.VMEM((_CHUNK, _DIM), jnp.float32),
        pltpu.SemaphoreType.DMA,
    ],
    compiler_params=pltpu.CompilerParams(use_tc_tiling_on_sc=False),
)
def _emb_lookup(pos_hbm, table_hbm, out_hbm, idx_v, rows_v, sem):
    wid = lax.axis_index("s") * _NC + lax.axis_index("c")
    base_row = wid * _ROWS_PER_W

    @pl.loop(0, _N_CHUNKS)
    def _chunk(i):
        irow = base_row + i * _GROUPS
        pltpu.sync_copy(pos_hbm.at[pl.ds(irow, _GROUPS), :], idx_v)
        copies = [
            pltpu.async_copy(
                table_hbm.at[idx_v.at[g]],
                rows_v.at[pl.ds(g * _IDX_COLS, _IDX_COLS), :],
                sem,
            )
            for g in range(_GROUPS)
        ]
        for c in copies:
            c.wait()
        pltpu.sync_copy(rows_v, out_hbm.at[pl.ds(irow * _IDX_COLS, _CHUNK), :])


def kernel(positions, table):
    pos_flat = positions.reshape(_IDX_ROWS, _IDX_COLS)
    out = _emb_lookup(pos_flat, table)
    return out.reshape(positions.shape[0], positions.shape[1], _DIM)
